# baseline (speedup 1.0000x reference)
import functools
import math

import jax
import jax.numpy as jnp
from jax import lax
from jax.experimental import pallas as pl
from jax.experimental.pallas import tpu as pltpu
from jax.experimental.pallas import tpu_sc as plsc

_LANES = 16


def _make_sc_kernel(N, D, NW, NC, CH, CPW, S):
    scale = math.sqrt(D)
    rows_per_w = N // NW
    nj = D // _LANES
    mesh = plsc.VectorSubcoreMesh(core_axis_name="c", subcore_axis_name="s")

    @functools.partial(
        pl.kernel,
        mesh=mesh,
        out_type=jax.ShapeDtypeStruct((N, D), jnp.float32),
        scratch_types=[
            pltpu.VMEM((CPW, CH), jnp.int32),
            pltpu.VMEM((2, CH, D), jnp.float32),
            pltpu.VMEM((2 * S, D), jnp.float32),
            pltpu.SemaphoreType.DMA,
            pltpu.SemaphoreType.DMA,
            pltpu.SemaphoreType.DMA,
            pltpu.SemaphoreType.DMA,
        ],
    )
    def k(w_hbm, idx_hbm, pe_hbm, out_hbm, idx_v, rows_v, pe_v, g0, g1, s0, s1):
        wid = lax.axis_index("s") * NC + lax.axis_index("c")
        base = wid * rows_per_w
        gsems = (g0, g1)
        ssems = (s0, s1)

        pltpu.sync_copy(pe_hbm, pe_v)
        pltpu.sync_copy(idx_hbm.at[wid], idx_v)
        pltpu.async_copy(w_hbm.at[idx_v.at[0]], rows_v.at[0], g0)

        def chunk_body(c, b):
            nb = 1 - b

            @pl.when(c >= 1)
            def _():
                pltpu.make_async_copy(
                    rows_v.at[nb],
                    out_hbm.at[pl.ds(base + (c - 1) * CH, CH)],
                    ssems[nb],
                ).wait()

            @pl.when(c + 1 < CPW)
            def _():
                pltpu.async_copy(
                    w_hbm.at[idx_v.at[c + 1]], rows_v.at[nb], gsems[nb]
                )

            pltpu.make_async_copy(
                w_hbm.at[idx_v.at[c]], rows_v.at[b], gsems[b]
            ).wait()

            p0 = lax.rem(c * CH, S)

            def inner(i, carry):
                for j in range(nj):
                    sl = pl.ds(j * _LANES, _LANES)
                    r = rows_v[b, i, sl]
                    p = pe_v[p0 + i, sl]
                    rows_v[b, i, sl] = r * scale + p
                return carry

            lax.fori_loop(0, CH, inner, 0, unroll=2)

            pltpu.async_copy(
                rows_v.at[b], out_hbm.at[pl.ds(base + c * CH, CH)], ssems[b]
            )

        def outer(t, carry):
            for b in range(2):
                chunk_body(t * 2 + b, b)
            return carry

        lax.fori_loop(0, CPW // 2, outer, 0)

        pltpu.make_async_copy(
            rows_v.at[1],
            out_hbm.at[pl.ds(base + (CPW - 1) * CH, CH)],
            s1,
        ).wait()

    return k


def kernel(input_seq, W, pos_emb):
    B, S = input_seq.shape
    _, D = W.shape
    N = B * S

    info = plsc.get_sparse_core_info()
    NC, NS = info.num_cores, info.num_subcores
    NW = NC * NS

    CH = 128
    assert D % _LANES == 0
    assert N % (NW * CH) == 0
    assert CH <= S
    CPW = N // (NW * CH)
    assert CPW % 2 == 0

    idx = input_seq.reshape(-1).astype(jnp.int32).reshape(NW, CPW, CH)
    pe = pos_emb[:S].astype(jnp.float32)
    pe2 = jnp.concatenate([pe, pe], axis=0)

    k = _make_sc_kernel(N, D, NW, NC, CH, CPW, S)
    out = k(W, idx, pe2)
    return out.reshape(B, S, D)


# speedup vs baseline: 2.5311x; 2.5311x over previous
import functools
import math

import jax
import jax.numpy as jnp
from jax import lax
from jax.experimental import pallas as pl
from jax.experimental.pallas import tpu as pltpu
from jax.experimental.pallas import tpu_sc as plsc

_LANES = 16


def _make_sc_kernel(N, D, NW, NC, CH, CPW, S):
    scale = math.sqrt(D)
    rows_per_w = N // NW
    nj = D // _LANES
    mesh = plsc.VectorSubcoreMesh(core_axis_name="c", subcore_axis_name="s")

    @functools.partial(
        pl.kernel,
        mesh=mesh,
        out_type=jax.ShapeDtypeStruct((N, D), jnp.float32),
        scratch_types=[
            pltpu.VMEM((CPW, CH), jnp.int32),
            pltpu.VMEM((2, CH, D), jnp.float32),
            pltpu.VMEM((2 * S, D), jnp.float32),
            pltpu.SemaphoreType.DMA,
            pltpu.SemaphoreType.DMA,
            pltpu.SemaphoreType.DMA,
            pltpu.SemaphoreType.DMA,
        ],
    )
    def k(w_hbm, idx_hbm, pe_hbm, out_hbm, idx_v, rows_v, pe_v, g0, g1, s0, s1):
        wid = lax.axis_index("s") * NC + lax.axis_index("c")
        base = wid * rows_per_w
        gsems = (g0, g1)
        ssems = (s0, s1)

        pltpu.sync_copy(pe_hbm, pe_v)
        pltpu.sync_copy(idx_hbm.at[wid], idx_v)
        pltpu.async_copy(w_hbm.at[idx_v.at[0]], rows_v.at[0], g0)

        def chunk_body(c, b):
            nb = 1 - b

            @pl.when(c >= 1)
            def _():
                pltpu.make_async_copy(
                    rows_v.at[nb],
                    out_hbm.at[pl.ds(base + (c - 1) * CH, CH)],
                    ssems[nb],
                ).wait()

            @pl.when(c + 1 < CPW)
            def _():
                pltpu.async_copy(
                    w_hbm.at[idx_v.at[c + 1]], rows_v.at[nb], gsems[nb]
                )

            pltpu.make_async_copy(
                w_hbm.at[idx_v.at[c]], rows_v.at[b], gsems[b]
            ).wait()

            p0 = lax.rem(c * CH, S)

            @plsc.parallel_loop(0, CH, unroll=4)
            def _(i):
                for j in range(nj):
                    sl = pl.ds(j * _LANES, _LANES)
                    r = rows_v[b, i, sl]
                    p = pe_v[p0 + i, sl]
                    rows_v[b, i, sl] = r * scale + p

            pltpu.async_copy(
                rows_v.at[b], out_hbm.at[pl.ds(base + c * CH, CH)], ssems[b]
            )

        def outer(t, carry):
            for b in range(2):
                chunk_body(t * 2 + b, b)
            return carry

        lax.fori_loop(0, CPW // 2, outer, 0)

        pltpu.make_async_copy(
            rows_v.at[1],
            out_hbm.at[pl.ds(base + (CPW - 1) * CH, CH)],
            s1,
        ).wait()

    return k


def kernel(input_seq, W, pos_emb):
    B, S = input_seq.shape
    _, D = W.shape
    N = B * S

    info = plsc.get_sparse_core_info()
    NC, NS = info.num_cores, info.num_subcores
    NW = NC * NS

    CH = 128
    assert D % _LANES == 0
    assert N % (NW * CH) == 0
    assert CH <= S
    CPW = N // (NW * CH)
    assert CPW % 2 == 0

    idx = input_seq.reshape(-1).astype(jnp.int32).reshape(NW, CPW, CH)
    pe = pos_emb[:S].astype(jnp.float32)
    pe2 = jnp.concatenate([pe, pe], axis=0)

    k = _make_sc_kernel(N, D, NW, NC, CH, CPW, S)
    out = k(W, idx, pe2)
    return out.reshape(B, S, D)


# speedup vs baseline: 2.8636x; 1.1314x over previous
import functools
import math

import jax
import jax.numpy as jnp
from jax import lax
from jax.experimental import pallas as pl
from jax.experimental.pallas import tpu as pltpu
from jax.experimental.pallas import tpu_sc as plsc

_LANES = 16


def _make_sc_kernel(N, D, NW, NC, S, G, QL, NQ, NGRP):
    scale = math.sqrt(D)
    nj = D // _LANES
    rows_per_tile = NGRP * G * QL
    mesh = plsc.VectorSubcoreMesh(core_axis_name="c", subcore_axis_name="s")

    @functools.partial(
        pl.kernel,
        mesh=mesh,
        out_type=jax.ShapeDtypeStruct((N, D), jnp.float32),
        scratch_types=[
            pltpu.VMEM((NGRP, G, QL), jnp.int32),
            pltpu.VMEM((2, G, QL, D), jnp.float32),
            pltpu.VMEM((S, D), jnp.float32),
            pltpu.SemaphoreType.DMA,
            pltpu.SemaphoreType.DMA,
            pltpu.SemaphoreType.DMA,
            pltpu.SemaphoreType.DMA,
        ],
    )
    def k(w_hbm, idx_hbm, pe_hbm, out_hbm, idx_v, rows_v, pe_v, g0, g1, s0, s1):
        wid = lax.axis_index("s") * NC + lax.axis_index("c")
        base = wid * rows_per_tile
        gsems = (g0, g1)
        ssems = (s0, s1)

        pltpu.sync_copy(pe_hbm, pe_v)
        pltpu.sync_copy(idx_hbm.at[wid], idx_v)

        def row0_of(g3, gg):
            q = g3 // (S // QL)
            h = lax.rem(g3, S // QL)
            return base + q * (G * S) + gg * S + h * QL

        def start_gathers(g3, b):
            for gg in range(G):
                pltpu.async_copy(
                    w_hbm.at[idx_v.at[g3, gg]], rows_v.at[b, gg], gsems[b]
                )

        def wait_gathers(g3, b):
            for gg in range(G):
                pltpu.make_async_copy(
                    w_hbm.at[idx_v.at[g3, gg]], rows_v.at[b, gg], gsems[b]
                ).wait()

        def start_stores(g3, b):
            for gg in range(G):
                pltpu.async_copy(
                    rows_v.at[b, gg],
                    out_hbm.at[pl.ds(row0_of(g3, gg), QL)],
                    ssems[b],
                )

        def wait_stores(g3, b):
            for gg in range(G):
                pltpu.make_async_copy(
                    rows_v.at[b, gg],
                    out_hbm.at[pl.ds(row0_of(g3, gg), QL)],
                    ssems[b],
                ).wait()

        start_gathers(0, 0)

        def group_body(g3, b):
            nb = 1 - b

            @pl.when(g3 >= 1)
            def _():
                wait_stores(g3 - 1, nb)

            @pl.when(g3 + 1 < NGRP)
            def _():
                start_gathers(g3 + 1, nb)

            wait_gathers(g3, b)

            p0 = lax.rem(g3, S // QL) * QL

            @plsc.parallel_loop(0, QL, unroll=2)
            def _(i):
                for j in range(nj):
                    sl = pl.ds(j * _LANES, _LANES)
                    p = pe_v[p0 + i, sl]
                    for gg in range(G):
                        r = rows_v[b, gg, i, sl]
                        rows_v[b, gg, i, sl] = r * scale + p

            start_stores(g3, b)

        def outer(t, carry):
            for b in range(2):
                group_body(t * 2 + b, b)
            return carry

        lax.fori_loop(0, NGRP // 2, outer, 0)

        wait_stores(NGRP - 1, 1)

    return k


def kernel(input_seq, W, pos_emb):
    B, S = input_seq.shape
    _, D = W.shape
    N = B * S

    info = plsc.get_sparse_core_info()
    NC, NS = info.num_cores, info.num_subcores
    NW = NC * NS

    G = 4
    QL = 40
    assert D % _LANES == 0
    assert S % QL == 0 and QL % 8 == 0
    assert B % (NW * G) == 0
    NQ = B // NW // G
    NGRP = NQ * (S // QL)
    assert NGRP % 2 == 0

    idx = input_seq.astype(jnp.int32).reshape(NW, NQ, G, S // QL, QL)
    idx = idx.transpose(0, 1, 3, 2, 4).reshape(NW, NGRP, G, QL)

    pe = pos_emb[:S].astype(jnp.float32)

    k = _make_sc_kernel(N, D, NW, NC, S, G, QL, NQ, NGRP)
    out = k(W, idx, pe)
    return out.reshape(B, S, D)


# speedup vs baseline: 3.0290x; 1.0577x over previous
import functools
import math

import jax
import jax.numpy as jnp
from jax import lax
from jax.experimental import pallas as pl
from jax.experimental.pallas import tpu as pltpu
from jax.experimental.pallas import tpu_sc as plsc

_LANES = 16


def _make_sc_kernel(N, D, NW, NC, S, G, QL, NGRP):
    scale = math.sqrt(D)
    nj = D // _LANES
    nh = S // QL
    rows_per_tile = NGRP * G * QL
    mesh = plsc.VectorSubcoreMesh(core_axis_name="c", subcore_axis_name="s")

    @functools.partial(
        pl.kernel,
        mesh=mesh,
        out_type=jax.ShapeDtypeStruct((N, D), jnp.float32),
        scratch_types=[
            pltpu.VMEM((rows_per_tile,), jnp.int32),
            pltpu.VMEM((2, G, QL, D), jnp.float32),
            pltpu.VMEM((S, D), jnp.float32),
            pltpu.SemaphoreType.DMA,
            pltpu.SemaphoreType.DMA,
            pltpu.SemaphoreType.DMA,
            pltpu.SemaphoreType.DMA,
        ],
    )
    def k(w_hbm, idx_hbm, pe_hbm, out_hbm, idx_v, rows_v, pe_v, g0, g1, s0, s1):
        wid = lax.axis_index("s") * NC + lax.axis_index("c")
        base = wid * rows_per_tile
        gsems = (g0, g1)
        ssems = (s0, s1)

        pltpu.sync_copy(pe_hbm, pe_v)
        pltpu.sync_copy(idx_hbm.at[pl.ds(base, rows_per_tile)], idx_v)

        def loc0_of(g3, gg):
            q = g3 // nh
            h = lax.rem(g3, nh)
            return q * (G * S) + gg * S + h * QL

        def start_gathers(g3, b):
            for gg in range(G):
                pltpu.async_copy(
                    w_hbm.at[idx_v.at[pl.ds(loc0_of(g3, gg), QL)]],
                    rows_v.at[b, gg],
                    gsems[b],
                )

        def wait_gathers(g3, b):
            for gg in range(G):
                pltpu.make_async_copy(
                    w_hbm.at[idx_v.at[pl.ds(loc0_of(g3, gg), QL)]],
                    rows_v.at[b, gg],
                    gsems[b],
                ).wait()

        def start_stores(g3, b):
            for gg in range(G):
                pltpu.async_copy(
                    rows_v.at[b, gg],
                    out_hbm.at[pl.ds(base + loc0_of(g3, gg), QL)],
                    ssems[b],
                )

        def wait_stores(g3, b):
            for gg in range(G):
                pltpu.make_async_copy(
                    rows_v.at[b, gg],
                    out_hbm.at[pl.ds(base + loc0_of(g3, gg), QL)],
                    ssems[b],
                ).wait()

        start_gathers(0, 0)

        def group_body(g3, b):
            nb = 1 - b

            @pl.when(g3 >= 1)
            def _():
                wait_stores(g3 - 1, nb)

            @pl.when(g3 + 1 < NGRP)
            def _():
                start_gathers(g3 + 1, nb)

            wait_gathers(g3, b)

            p0 = lax.rem(g3, nh) * QL

            @plsc.parallel_loop(0, QL, unroll=2)
            def _(i):
                for j in range(nj):
                    sl = pl.ds(j * _LANES, _LANES)
                    p = pe_v[p0 + i, sl]
                    for gg in range(G):
                        r = rows_v[b, gg, i, sl]
                        rows_v[b, gg, i, sl] = r * scale + p

            start_stores(g3, b)

        def outer(t, carry):
            for b in range(2):
                group_body(t * 2 + b, b)
            return carry

        lax.fori_loop(0, NGRP // 2, outer, 0)

        wait_stores(NGRP - 1, 1)

    return k


def kernel(input_seq, W, pos_emb):
    B, S = input_seq.shape
    _, D = W.shape
    N = B * S

    info = plsc.get_sparse_core_info()
    NC, NS = info.num_cores, info.num_subcores
    NW = NC * NS

    G = 8
    QL = 40
    assert D % _LANES == 0
    assert S % QL == 0 and QL % 8 == 0
    assert B % (NW * G) == 0
    NGRP = (B // NW // G) * (S // QL)
    assert NGRP % 2 == 0

    idx = input_seq.astype(jnp.int32).reshape(-1)
    pe = pos_emb[:S].astype(jnp.float32)

    k = _make_sc_kernel(N, D, NW, NC, S, G, QL, NGRP)
    out = k(W, idx, pe)
    return out.reshape(B, S, D)
